# baseline (speedup 1.0000x reference)
import functools

import jax
import jax.numpy as jnp
from jax import lax
from jax.experimental import pallas as pl
from jax.experimental.pallas import tpu as pltpu
from jax.experimental.pallas import tpu_sc as plsc

TOP_K = 2
NOISE_EPS = 0.01

_SC_INFO = plsc.get_sparse_core_info()
_NC, _NS, _L = _SC_INFO.num_cores, _SC_INFO.num_subcores, _SC_INFO.num_lanes
_NW = _NC * _NS


@functools.lru_cache(maxsize=None)
def _noise_const(n, e):
    return jnp.transpose(
        jax.random.normal(jax.random.key(42), (n, e), dtype=jnp.float32))


def _logits_kernel(hs_ref, w_ref, noise_ref, sumv_ref, router_ref, stddev_ref,
                   *, e):
    logits = lax.dot_general(
        w_ref[...], hs_ref[...], (((1,), (1,)), ((), ())),
        preferred_element_type=jnp.float32)
    router = logits[:e, :]
    noise_logits = logits[e:, :]
    stddev = jnp.log1p(jnp.exp(noise_logits)) + NOISE_EPS
    router_ref[...] = router
    stddev_ref[...] = stddev
    sumv_ref[...] = router + noise_ref[...] * stddev


def _sc_router(sumv_hbm, rw0_hbm, rw1_hbm, sel0_hbm, sel1_hbm, l1_hbm, l2_hbm,
               buf, orw0, orw1, osel0, osel1, ol1, ol2, *, tpw, e):
    wid = lax.axis_index("s") * _NC + lax.axis_index("c")
    base = wid * tpw
    pltpu.sync_copy(sumv_hbm.at[:, pl.ds(base, tpw)], buf)

    def group(g, carry):
        off = g * _L
        neg_inf = jnp.full((_L,), -jnp.inf, jnp.float32)
        m0 = neg_inf
        m1 = neg_inf
        m2 = neg_inf
        i0 = jnp.zeros((_L,), jnp.int32)
        i1 = jnp.zeros((_L,), jnp.int32)
        for ee in range(e):
            v = buf[ee, pl.ds(off, _L)]
            c0 = v > m0
            c1 = v > m1
            c2 = v > m2
            en = jnp.full((_L,), ee, jnp.int32)
            i1 = jnp.where(c0, i0, jnp.where(c1, en, i1))
            i0 = jnp.where(c0, en, i0)
            m2 = jnp.where(c0 | c1, m1, jnp.where(c2, v, m2))
            m1 = jnp.where(c0, m0, jnp.where(c1, v, m1))
            m0 = jnp.where(c0, v, m0)
        ex = jnp.exp(m1 - m0)
        r = 1.0 / (1.0 + ex)
        orw0[pl.ds(off, _L)] = r
        orw1[pl.ds(off, _L)] = ex * r
        osel0[pl.ds(off, _L)] = i0
        osel1[pl.ds(off, _L)] = i1
        ol1[pl.ds(off, _L)] = m1
        ol2[pl.ds(off, _L)] = m2
        return carry

    lax.fori_loop(0, tpw // _L, group, 0)

    pltpu.sync_copy(orw0, rw0_hbm.at[pl.ds(base, tpw)])
    pltpu.sync_copy(orw1, rw1_hbm.at[pl.ds(base, tpw)])
    pltpu.sync_copy(osel0, sel0_hbm.at[pl.ds(base, tpw)])
    pltpu.sync_copy(osel1, sel1_hbm.at[pl.ds(base, tpw)])
    pltpu.sync_copy(ol1, l1_hbm.at[pl.ds(base, tpw)])
    pltpu.sync_copy(ol2, l2_hbm.at[pl.ds(base, tpw)])


def _post_kernel(sumv_ref, router_ref, stddev_ref, l1_ref, l2_ref,
                 s0_ref, s1_ref, mask_ref, load_ref, *, bt, e):
    sumv = sumv_ref[...]
    l1 = l1_ref[...]
    l2 = l2_ref[...]
    thr = jnp.where(sumv > l2, l2, l1)
    scale = jnp.float32(0.7071067811865475) / stddev_ref[...]
    load_ref[...] = 0.5 * (1.0 + lax.erf((router_ref[...] - thr) * scale))

    sel_k = jnp.concatenate(
        [s0_ref[...][:, None, :], s1_ref[...][:, None, :]], axis=1)
    e_iota3 = lax.broadcasted_iota(jnp.int32, (e, TOP_K, bt), 0)
    mask_ref[...] = (e_iota3 == sel_k).astype(jnp.int32)


def kernel(x, W_route, W_noise):
    b, s, d = x.shape
    n = b * s
    e = W_route.shape[0]
    hs = x.reshape(n, d)
    w = jnp.concatenate([W_route, W_noise], axis=0)
    noise = _noise_const(n, e)

    bt = 1024 if n % 1024 == 0 else n
    grid = (n // bt,)

    body_a = functools.partial(_logits_kernel, e=e)
    sumv, router, stddev = pl.pallas_call(
        body_a,
        grid=grid,
        in_specs=[
            pl.BlockSpec((bt, d), lambda i: (i, 0)),
            pl.BlockSpec((2 * e, d), lambda i: (0, 0)),
            pl.BlockSpec((e, bt), lambda i: (0, i)),
        ],
        out_specs=[
            pl.BlockSpec((e, bt), lambda i: (0, i)),
            pl.BlockSpec((e, bt), lambda i: (0, i)),
            pl.BlockSpec((e, bt), lambda i: (0, i)),
        ],
        out_shape=[
            jax.ShapeDtypeStruct((e, n), jnp.float32),
            jax.ShapeDtypeStruct((e, n), jnp.float32),
            jax.ShapeDtypeStruct((e, n), jnp.float32),
        ],
        compiler_params=pltpu.CompilerParams(
            dimension_semantics=(pltpu.PARALLEL,),
            vmem_limit_bytes=100 * 1024 * 1024,
        ),
    )(hs, w, noise)

    tpw = n // _NW
    mesh = plsc.VectorSubcoreMesh(core_axis_name="c", subcore_axis_name="s")
    sc_body = functools.partial(_sc_router, tpw=tpw, e=e)
    sc_call = pl.kernel(
        sc_body,
        out_type=[
            jax.ShapeDtypeStruct((n,), jnp.float32),
            jax.ShapeDtypeStruct((n,), jnp.float32),
            jax.ShapeDtypeStruct((n,), jnp.int32),
            jax.ShapeDtypeStruct((n,), jnp.int32),
            jax.ShapeDtypeStruct((n,), jnp.float32),
            jax.ShapeDtypeStruct((n,), jnp.float32),
        ],
        mesh=mesh,
        scratch_types=[
            pltpu.VMEM((e, tpw), jnp.float32),
            pltpu.VMEM((tpw,), jnp.float32),
            pltpu.VMEM((tpw,), jnp.float32),
            pltpu.VMEM((tpw,), jnp.int32),
            pltpu.VMEM((tpw,), jnp.int32),
            pltpu.VMEM((tpw,), jnp.float32),
            pltpu.VMEM((tpw,), jnp.float32),
        ],
    )
    rw0, rw1, sel0, sel1, l1, l2 = sc_call(sumv)

    body_c = functools.partial(_post_kernel, bt=bt, e=e)
    mask, load_t = pl.pallas_call(
        body_c,
        grid=grid,
        in_specs=[
            pl.BlockSpec((e, bt), lambda i: (0, i)),
            pl.BlockSpec((e, bt), lambda i: (0, i)),
            pl.BlockSpec((e, bt), lambda i: (0, i)),
            pl.BlockSpec((1, bt), lambda i: (0, i)),
            pl.BlockSpec((1, bt), lambda i: (0, i)),
            pl.BlockSpec((1, bt), lambda i: (0, i)),
            pl.BlockSpec((1, bt), lambda i: (0, i)),
        ],
        out_specs=[
            pl.BlockSpec((e, TOP_K, bt), lambda i: (0, 0, i)),
            pl.BlockSpec((e, bt), lambda i: (0, i)),
        ],
        out_shape=[
            jax.ShapeDtypeStruct((e, TOP_K, n), jnp.int32),
            jax.ShapeDtypeStruct((e, n), jnp.float32),
        ],
        compiler_params=pltpu.CompilerParams(
            dimension_semantics=(pltpu.PARALLEL,),
            vmem_limit_bytes=100 * 1024 * 1024,
        ),
    )(sumv, router, stddev,
      l1.reshape(1, n), l2.reshape(1, n),
      sel0.reshape(1, n), sel1.reshape(1, n))

    rw = jnp.concatenate([rw0[:, None], rw1[:, None]], axis=1)
    sel = jnp.concatenate([sel0[:, None], sel1[:, None]], axis=1)
    return (rw, sel, mask, jnp.transpose(load_t))


# speedup vs baseline: 1.0065x; 1.0065x over previous
import functools

import jax
import jax.numpy as jnp
from jax import lax
from jax.experimental import pallas as pl
from jax.experimental.pallas import tpu as pltpu
from jax.experimental.pallas import tpu_sc as plsc

TOP_K = 2
NOISE_EPS = 0.01

_SC_INFO = plsc.get_sparse_core_info()
_NC, _NS, _L = _SC_INFO.num_cores, _SC_INFO.num_subcores, _SC_INFO.num_lanes
_NW = _NC * _NS


@functools.lru_cache(maxsize=None)
def _noise_const(n, e):
    return jnp.transpose(
        jax.random.normal(jax.random.key(42), (n, e), dtype=jnp.float32))


def _logits_kernel(hs_ref, w_ref, noise_ref, sumv_ref, router_ref, stddev_ref,
                   *, e):
    logits = lax.dot_general(
        w_ref[...], hs_ref[...], (((1,), (1,)), ((), ())),
        preferred_element_type=jnp.float32)
    router = logits[:e, :]
    noise_logits = logits[e:, :]
    stddev = jnp.log1p(jnp.exp(noise_logits)) + NOISE_EPS
    router_ref[...] = router
    stddev_ref[...] = stddev
    sumv_ref[...] = router + noise_ref[...] * stddev


def _sc_router(sumv_hbm, rw0_hbm, rw1_hbm, sel0_hbm, sel1_hbm, l1_hbm, l2_hbm,
               buf, orw0, orw1, osel0, osel1, ol1, ol2, *, tpw, e):
    wid = lax.axis_index("s") * _NC + lax.axis_index("c")
    base = wid * tpw
    pltpu.sync_copy(sumv_hbm.at[:, pl.ds(base, tpw)], buf)

    def group(g, carry):
        off = g * _L
        neg_inf = jnp.full((_L,), -jnp.inf, jnp.float32)
        m0 = neg_inf
        m1 = neg_inf
        m2 = neg_inf
        i0 = jnp.zeros((_L,), jnp.int32)
        i1 = jnp.zeros((_L,), jnp.int32)
        for ee in range(e):
            v = buf[ee, pl.ds(off, _L)]
            c0 = v > m0
            c1 = v > m1
            c2 = v > m2
            en = jnp.full((_L,), ee, jnp.int32)
            i1 = jnp.where(c0, i0, jnp.where(c1, en, i1))
            i0 = jnp.where(c0, en, i0)
            m2 = jnp.where(c0 | c1, m1, jnp.where(c2, v, m2))
            m1 = jnp.where(c0, m0, jnp.where(c1, v, m1))
            m0 = jnp.where(c0, v, m0)
        ex = jnp.exp(m1 - m0)
        r = 1.0 / (1.0 + ex)
        orw0[pl.ds(off, _L)] = r
        orw1[pl.ds(off, _L)] = ex * r
        osel0[pl.ds(off, _L)] = i0
        osel1[pl.ds(off, _L)] = i1
        ol1[pl.ds(off, _L)] = m1
        ol2[pl.ds(off, _L)] = m2
        return carry

    lax.fori_loop(0, tpw // _L, group, 0)

    pltpu.sync_copy(orw0, rw0_hbm.at[pl.ds(base, tpw)])
    pltpu.sync_copy(orw1, rw1_hbm.at[pl.ds(base, tpw)])
    pltpu.sync_copy(osel0, sel0_hbm.at[pl.ds(base, tpw)])
    pltpu.sync_copy(osel1, sel1_hbm.at[pl.ds(base, tpw)])
    pltpu.sync_copy(ol1, l1_hbm.at[pl.ds(base, tpw)])
    pltpu.sync_copy(ol2, l2_hbm.at[pl.ds(base, tpw)])


def _post_kernel(router_ref, stddev_ref, l1_ref, l2_ref,
                 s0_ref, s1_ref, mask_ref, load_ref, *, bt, e):
    l1 = l1_ref[...]
    l2 = l2_ref[...]
    e_iota = lax.broadcasted_iota(jnp.int32, (e, bt), 0)
    in_top2 = (e_iota == s0_ref[...]) | (e_iota == s1_ref[...])
    thr = jnp.where(in_top2, l2, l1)
    scale = jnp.float32(0.7071067811865475) / stddev_ref[...]
    load_ref[...] = 0.5 * (1.0 + lax.erf((router_ref[...] - thr) * scale))

    sel_k = jnp.concatenate(
        [s0_ref[...][:, None, :], s1_ref[...][:, None, :]], axis=1)
    e_iota3 = lax.broadcasted_iota(jnp.int32, (e, TOP_K, bt), 0)
    mask_ref[...] = (e_iota3 == sel_k).astype(jnp.int32)


def kernel(x, W_route, W_noise):
    b, s, d = x.shape
    n = b * s
    e = W_route.shape[0]
    hs = x.reshape(n, d)
    w = jnp.concatenate([W_route, W_noise], axis=0)
    noise = _noise_const(n, e)

    bt = 1024 if n % 1024 == 0 else n
    grid = (n // bt,)

    body_a = functools.partial(_logits_kernel, e=e)
    sumv, router, stddev = pl.pallas_call(
        body_a,
        grid=grid,
        in_specs=[
            pl.BlockSpec((bt, d), lambda i: (i, 0)),
            pl.BlockSpec((2 * e, d), lambda i: (0, 0)),
            pl.BlockSpec((e, bt), lambda i: (0, i)),
        ],
        out_specs=[
            pl.BlockSpec((e, bt), lambda i: (0, i)),
            pl.BlockSpec((e, bt), lambda i: (0, i)),
            pl.BlockSpec((e, bt), lambda i: (0, i)),
        ],
        out_shape=[
            jax.ShapeDtypeStruct((e, n), jnp.float32),
            jax.ShapeDtypeStruct((e, n), jnp.float32),
            jax.ShapeDtypeStruct((e, n), jnp.float32),
        ],
        compiler_params=pltpu.CompilerParams(
            dimension_semantics=(pltpu.PARALLEL,),
            vmem_limit_bytes=100 * 1024 * 1024,
        ),
    )(hs, w, noise)

    tpw = n // _NW
    mesh = plsc.VectorSubcoreMesh(core_axis_name="c", subcore_axis_name="s")
    sc_body = functools.partial(_sc_router, tpw=tpw, e=e)
    sc_call = pl.kernel(
        sc_body,
        out_type=[
            jax.ShapeDtypeStruct((n,), jnp.float32),
            jax.ShapeDtypeStruct((n,), jnp.float32),
            jax.ShapeDtypeStruct((n,), jnp.int32),
            jax.ShapeDtypeStruct((n,), jnp.int32),
            jax.ShapeDtypeStruct((n,), jnp.float32),
            jax.ShapeDtypeStruct((n,), jnp.float32),
        ],
        mesh=mesh,
        scratch_types=[
            pltpu.VMEM((e, tpw), jnp.float32),
            pltpu.VMEM((tpw,), jnp.float32),
            pltpu.VMEM((tpw,), jnp.float32),
            pltpu.VMEM((tpw,), jnp.int32),
            pltpu.VMEM((tpw,), jnp.int32),
            pltpu.VMEM((tpw,), jnp.float32),
            pltpu.VMEM((tpw,), jnp.float32),
        ],
    )
    rw0, rw1, sel0, sel1, l1, l2 = sc_call(sumv)

    body_c = functools.partial(_post_kernel, bt=bt, e=e)
    mask, load_t = pl.pallas_call(
        body_c,
        grid=grid,
        in_specs=[
            pl.BlockSpec((e, bt), lambda i: (0, i)),
            pl.BlockSpec((e, bt), lambda i: (0, i)),
            pl.BlockSpec((1, bt), lambda i: (0, i)),
            pl.BlockSpec((1, bt), lambda i: (0, i)),
            pl.BlockSpec((1, bt), lambda i: (0, i)),
            pl.BlockSpec((1, bt), lambda i: (0, i)),
        ],
        out_specs=[
            pl.BlockSpec((e, TOP_K, bt), lambda i: (0, 0, i)),
            pl.BlockSpec((e, bt), lambda i: (0, i)),
        ],
        out_shape=[
            jax.ShapeDtypeStruct((e, TOP_K, n), jnp.int32),
            jax.ShapeDtypeStruct((e, n), jnp.float32),
        ],
        compiler_params=pltpu.CompilerParams(
            dimension_semantics=(pltpu.PARALLEL,),
            vmem_limit_bytes=100 * 1024 * 1024,
        ),
    )(router, stddev,
      l1.reshape(1, n), l2.reshape(1, n),
      sel0.reshape(1, n), sel1.reshape(1, n))

    rw = jnp.concatenate([rw0[:, None], rw1[:, None]], axis=1)
    sel = jnp.concatenate([sel0[:, None], sel1[:, None]], axis=1)
    return (rw, sel, mask, jnp.transpose(load_t))
